# Initial kernel scaffold; baseline (speedup 1.0000x reference)
import jax
import jax.numpy as jnp
from jax.experimental import pallas as pl


def kernel(node_features, edges, input_node_indices, W1, b1, Wp0, bp0, Wu0, bu0, Wp1, bp1, Wu1, bu1, W2, b2, W3, b3):
    raise NotImplementedError("write your pallas kernel here")


# speedup vs baseline: 6.6448x; 6.6448x over previous
import functools

import jax
import jax.numpy as jnp
from jax import lax
from jax.experimental import pallas as pl
from jax.experimental.pallas import tpu as pltpu
from jax.experimental.pallas import tpu_sc as plsc

N = 10000
E = 320000
D = 128
H = 128
C = 7
B = 4096

NC = 2
NS = 16
NW = NC * NS
NPAD = 10240
ROWS_PER_TILE = NPAD // NS
K = 128
EDGES_PER_TILE = E // NW
CHUNKS_PER_TILE = EDGES_PER_TILE // K
K_TAIL = EDGES_PER_TILE - CHUNKS_PER_TILE * K

_mesh = lambda: plsc.VectorSubcoreMesh(
    core_axis_name="c", subcore_axis_name="s", num_cores=NC, num_subcores=NS)


def _gelu(x):
    return jax.nn.gelu(x)


_RB = 1000


def _dense2_body(x_ref, wa_ref, ba_ref, wb_ref, bb_ref, ya_ref, yb_ref):
    ya = _gelu(jnp.dot(x_ref[...], wa_ref[...],
                       preferred_element_type=jnp.float32) + ba_ref[...])
    ya_ref[...] = ya
    yb_ref[...] = _gelu(jnp.dot(ya, wb_ref[...],
                                preferred_element_type=jnp.float32) + bb_ref[...])


def _tc_dense2(x, wa, ba, wb, bb):
    n = x.shape[0]
    grid = (n // _RB,)
    full = lambda i: (0, 0)
    return pl.pallas_call(
        _dense2_body,
        grid=grid,
        in_specs=[
            pl.BlockSpec((_RB, H), lambda i: (i, 0)),
            pl.BlockSpec((H, H), full), pl.BlockSpec((1, H), full),
            pl.BlockSpec((H, H), full), pl.BlockSpec((1, H), full),
        ],
        out_specs=[pl.BlockSpec((_RB, H), lambda i: (i, 0)),
                   pl.BlockSpec((_RB, H), lambda i: (i, 0))],
        out_shape=[jax.ShapeDtypeStruct((n, H), jnp.float32),
                   jax.ShapeDtypeStruct((n, H), jnp.float32)],
    )(x, wa, ba, wb, bb)


def _update_body(with_prep, agg_ref, deg_ref, x_ref, wu_ref, bu_ref,
                 wp_ref, bp_ref, *out_refs):
    deg = jnp.maximum(deg_ref[0, :, 0:1] + deg_ref[1, :, 0:1], 1.0)
    a = (agg_ref[0] + agg_ref[1]) * (1.0 / deg)
    x2 = _gelu(jnp.dot(a, wu_ref[...],
                       preferred_element_type=jnp.float32) + bu_ref[...]) + x_ref[...]
    out_refs[0][...] = x2
    if with_prep:
        out_refs[1][...] = _gelu(jnp.dot(x2, wp_ref[...],
                                         preferred_element_type=jnp.float32) + bp_ref[...])


def _tc_update(agg, deg, x, wu, bu, wp, bp, with_prep):
    grid = (N // _RB,)
    full = lambda i: (0, 0)
    out_specs = [pl.BlockSpec((_RB, H), lambda i: (i, 0))]
    out_shape = [jax.ShapeDtypeStruct((N, H), jnp.float32)]
    if with_prep:
        out_specs = out_specs * 2
        out_shape = out_shape * 2
    res = pl.pallas_call(
        functools.partial(_update_body, with_prep),
        grid=grid,
        in_specs=[
            pl.BlockSpec((2, _RB, H), lambda i: (0, i, 0)),
            pl.BlockSpec((2, _RB, 1), lambda i: (0, i, 0)),
            pl.BlockSpec((_RB, H), lambda i: (i, 0)),
            pl.BlockSpec((H, H), full), pl.BlockSpec((1, H), full),
            pl.BlockSpec((H, H), full), pl.BlockSpec((1, H), full),
        ],
        out_specs=out_specs,
        out_shape=out_shape,
    )(agg, deg, x, wu, bu, wp, bp)
    return res if with_prep else (res[0], None)


def _head_body(g_ref, w2_ref, b2_ref, w3_ref, b3_ref, o_ref):
    t = _gelu(jnp.dot(g_ref[...], w2_ref[...],
                      preferred_element_type=jnp.float32) + b2_ref[...])
    o_ref[...] = jnp.dot(t, w3_ref[...],
                         preferred_element_type=jnp.float32) + b3_ref[...]


def _tc_head(g, w2, b2, w3, b3):
    rb = 512
    grid = (B // rb,)
    full = lambda i: (0, 0)
    return pl.pallas_call(
        _head_body,
        grid=grid,
        in_specs=[
            pl.BlockSpec((rb, H), lambda i: (i, 0)),
            pl.BlockSpec((H, H), full), pl.BlockSpec((1, H), full),
            pl.BlockSpec((H, C), full), pl.BlockSpec((1, C), full),
        ],
        out_specs=pl.BlockSpec((rb, C), lambda i: (i, 0)),
        out_shape=jax.ShapeDtypeStruct((B, C), jnp.float32),
    )(g, w2, b2, w3, b3)


def _zero_vmem_rows(ref, nrows, ncols):
    z = jnp.zeros((16,), jnp.float32)

    def body(i, _):
        r = i // (ncols // 16)
        c = (i % (ncols // 16)) * 16
        ref[r, pl.ds(c, 16)] = z
        return ()

    lax.fori_loop(0, nrows * (ncols // 16), body, ())


def _edge_agg_body(h_hbm, src_hbm, dst_hbm, agg_hbm,
                   agg_sh, sidx_v, didx_v, rows_v, sidx_t, didx_t):
    cid = lax.axis_index("c")
    sid = lax.axis_index("s")

    _zero_vmem_rows(rows_v, K, H)
    for j in range(ROWS_PER_TILE // K):
        pltpu.sync_copy(rows_v, agg_sh.at[pl.ds(sid * ROWS_PER_TILE + j * K, K)])
    plsc.subcore_barrier()

    base = (cid * NS + sid) * EDGES_PER_TILE

    def chunk(i, _):
        off = base + i * K
        pltpu.sync_copy(src_hbm.at[pl.ds(off, K)], sidx_v)
        pltpu.sync_copy(dst_hbm.at[pl.ds(off, K)], didx_v)
        pltpu.sync_copy(h_hbm.at[sidx_v], rows_v)
        pltpu.sync_copy(rows_v, agg_sh.at[didx_v], add=True)
        return ()

    lax.fori_loop(0, CHUNKS_PER_TILE, chunk, ())
    if K_TAIL:
        off = base + CHUNKS_PER_TILE * K
        pltpu.sync_copy(src_hbm.at[pl.ds(off, K_TAIL)], sidx_t)
        pltpu.sync_copy(dst_hbm.at[pl.ds(off, K_TAIL)], didx_t)
        pltpu.sync_copy(h_hbm.at[sidx_t], rows_v.at[pl.ds(0, K_TAIL)])
        pltpu.sync_copy(rows_v.at[pl.ds(0, K_TAIL)], agg_sh.at[didx_t], add=True)
    plsc.subcore_barrier()

    for j in range(ROWS_PER_TILE // K):
        r0 = sid * ROWS_PER_TILE + j * K
        pltpu.sync_copy(agg_sh.at[pl.ds(r0, K)], rows_v)
        pltpu.sync_copy(rows_v, agg_hbm.at[cid, pl.ds(r0, K)])


def _sc_edge_agg(h, src, dst):
    fn = pl.kernel(
        _edge_agg_body,
        out_type=jax.ShapeDtypeStruct((NC, NPAD, H), jnp.float32),
        mesh=_mesh(),
        compiler_params=pltpu.CompilerParams(needs_layout_passes=False),
        scratch_types=(
            pltpu.VMEM_SHARED((NPAD, H), jnp.float32),
            pltpu.VMEM((K,), jnp.int32),
            pltpu.VMEM((K,), jnp.int32),
            pltpu.VMEM((K, H), jnp.float32),
            pltpu.VMEM((K_TAIL,), jnp.int32),
            pltpu.VMEM((K_TAIL,), jnp.int32),
        ),
    )
    return fn(h, src, dst)


_HR = 128
_HR_PER_TILE = _HR // NS


def _degree_body(dst_hbm, deg_hbm, stage_sh, didx_v, hist_v, acc_v, tmp_v):
    cid = lax.axis_index("c")
    sid = lax.axis_index("s")
    z = jnp.zeros((16,), jnp.float32)

    def zrow(i, _):
        hist_v[i // 8, pl.ds((i % 8) * 16, 16)] = z
        return ()

    lax.fori_loop(0, _HR * 8, zrow, ())

    base = (cid * NS + sid) * EDGES_PER_TILE
    pltpu.sync_copy(dst_hbm.at[pl.ds(base, EDGES_PER_TILE)], didx_v)
    one = jnp.ones((16,), jnp.float32)

    def step(j, _):
        idx = didx_v[pl.ds(j * 16, 16)]
        plsc.addupdate_scatter(
            hist_v, [lax.shift_right_logical(idx, 7),
                     lax.bitwise_and(idx, 127)], one)
        return ()

    lax.fori_loop(0, EDGES_PER_TILE // 16, step, ())

    pltpu.sync_copy(hist_v, stage_sh.at[sid])
    plsc.subcore_barrier()
    r0 = sid * _HR_PER_TILE

    def zacc(i, _):
        acc_v[i // 8, pl.ds((i % 8) * 16, 16)] = z
        return ()

    lax.fori_loop(0, _HR_PER_TILE * 8, zacc, ())
    for t in range(NS):
        pltpu.sync_copy(stage_sh.at[t, pl.ds(r0, _HR_PER_TILE)], tmp_v)

        def radd(i, _):
            r, c = i // 8, (i % 8) * 16
            acc_v[r, pl.ds(c, 16)] += tmp_v[r, pl.ds(c, 16)]
            return ()

        lax.fori_loop(0, _HR_PER_TILE * 8, radd, ())
    pltpu.sync_copy(acc_v, deg_hbm.at[cid, pl.ds(r0, _HR_PER_TILE)])


def _sc_degree(dst):
    fn = pl.kernel(
        _degree_body,
        out_type=jax.ShapeDtypeStruct((NC, _HR, 128), jnp.float32),
        mesh=_mesh(),
        compiler_params=pltpu.CompilerParams(needs_layout_passes=False),
        scratch_types=(
            pltpu.VMEM_SHARED((NS, _HR, 128), jnp.float32),
            pltpu.VMEM((EDGES_PER_TILE,), jnp.int32),
            pltpu.VMEM((_HR, 128), jnp.float32),
            pltpu.VMEM((_HR_PER_TILE, 128), jnp.float32),
            pltpu.VMEM((_HR_PER_TILE, 128), jnp.float32),
        ),
    )
    return fn(dst).reshape(NC, _HR * 128, 1)


def _gather_body(x_hbm, idx_hbm, out_hbm, idx_v, rows_v):
    wid = lax.axis_index("c") * NS + lax.axis_index("s")
    b0 = wid * (B // NW)
    pltpu.sync_copy(idx_hbm.at[pl.ds(b0, B // NW)], idx_v)
    pltpu.sync_copy(x_hbm.at[idx_v], rows_v)
    pltpu.sync_copy(rows_v, out_hbm.at[pl.ds(b0, B // NW)])


def _sc_gather(x, idx):
    fn = pl.kernel(
        _gather_body,
        out_type=jax.ShapeDtypeStruct((B, H), jnp.float32),
        mesh=_mesh(),
        compiler_params=pltpu.CompilerParams(needs_layout_passes=False),
        scratch_types=(
            pltpu.VMEM((B // NW,), jnp.int32),
            pltpu.VMEM((B // NW, H), jnp.float32),
        ),
    )
    return fn(x, idx)


def kernel(node_features, edges, input_node_indices,
           W1, b1, Wp0, bp0, Wu0, bu0, Wp1, bp1, Wu1, bu1,
           W2, b2, W3, b3):
    src = edges[0]
    dst = edges[1]
    r = lambda b: b.reshape(1, -1)

    x1, h0 = _tc_dense2(node_features, W1, r(b1), Wp0, r(bp0))
    deg = _sc_degree(dst)
    agg0 = _sc_edge_agg(h0, src, dst)
    x2, h1 = _tc_update(agg0, deg, x1, Wu0, r(bu0), Wp1, r(bp1), with_prep=True)
    agg1 = _sc_edge_agg(h1, src, dst)
    x3, _ = _tc_update(agg1, deg, x2, Wu1, r(bu1), Wu1, r(bu1), with_prep=False)
    g = _sc_gather(x3, input_node_indices)
    return _tc_head(g, W2, r(b2), W3, r(b3))


# speedup vs baseline: 9.3216x; 1.4028x over previous
import functools

import jax
import jax.numpy as jnp
from jax import lax
from jax.experimental import pallas as pl
from jax.experimental.pallas import tpu as pltpu
from jax.experimental.pallas import tpu_sc as plsc

N = 10000
E = 320000
D = 128
H = 128
C = 7
B = 4096

NC = 2
NS = 16
NW = NC * NS
NPAD = 10240
ROWS_PER_TILE = NPAD // NS
K = 128
EDGES_PER_TILE = E // NW
CHUNKS_PER_TILE = EDGES_PER_TILE // K
K_TAIL = EDGES_PER_TILE - CHUNKS_PER_TILE * K

_mesh = lambda: plsc.VectorSubcoreMesh(
    core_axis_name="c", subcore_axis_name="s", num_cores=NC, num_subcores=NS)


def _gelu(x):
    return jax.nn.gelu(x)


_RB = 1000


def _dense2_body(x_ref, wa_ref, ba_ref, wb_ref, bb_ref, ya_ref, yb_ref):
    ya = _gelu(jnp.dot(x_ref[...], wa_ref[...],
                       preferred_element_type=jnp.float32) + ba_ref[...])
    ya_ref[...] = ya
    yb_ref[...] = _gelu(jnp.dot(ya, wb_ref[...],
                                preferred_element_type=jnp.float32) + bb_ref[...])


def _tc_dense2(x, wa, ba, wb, bb):
    n = x.shape[0]
    grid = (n // _RB,)
    full = lambda i: (0, 0)
    return pl.pallas_call(
        _dense2_body,
        grid=grid,
        in_specs=[
            pl.BlockSpec((_RB, H), lambda i: (i, 0)),
            pl.BlockSpec((H, H), full), pl.BlockSpec((1, H), full),
            pl.BlockSpec((H, H), full), pl.BlockSpec((1, H), full),
        ],
        out_specs=[pl.BlockSpec((_RB, H), lambda i: (i, 0)),
                   pl.BlockSpec((_RB, H), lambda i: (i, 0))],
        out_shape=[jax.ShapeDtypeStruct((n, H), jnp.float32),
                   jax.ShapeDtypeStruct((n, H), jnp.float32)],
    )(x, wa, ba, wb, bb)


def _update_body(with_prep, agg_ref, deg_ref, x_ref, wu_ref, bu_ref,
                 wp_ref, bp_ref, *out_refs):
    deg = jnp.maximum(deg_ref[0, :, 0:1] + deg_ref[1, :, 0:1], 1.0)
    a = (agg_ref[0] + agg_ref[1]) * (1.0 / deg)
    x2 = _gelu(jnp.dot(a, wu_ref[...],
                       preferred_element_type=jnp.float32) + bu_ref[...]) + x_ref[...]
    out_refs[0][...] = x2
    if with_prep:
        out_refs[1][...] = _gelu(jnp.dot(x2, wp_ref[...],
                                         preferred_element_type=jnp.float32) + bp_ref[...])


def _tc_update(agg, deg, x, wu, bu, wp, bp, with_prep):
    grid = (N // _RB,)
    full = lambda i: (0, 0)
    out_specs = [pl.BlockSpec((_RB, H), lambda i: (i, 0))]
    out_shape = [jax.ShapeDtypeStruct((N, H), jnp.float32)]
    if with_prep:
        out_specs = out_specs * 2
        out_shape = out_shape * 2
    res = pl.pallas_call(
        functools.partial(_update_body, with_prep),
        grid=grid,
        in_specs=[
            pl.BlockSpec((2, _RB, H), lambda i: (0, i, 0)),
            pl.BlockSpec((2, _RB, 1), lambda i: (0, i, 0)),
            pl.BlockSpec((_RB, H), lambda i: (i, 0)),
            pl.BlockSpec((H, H), full), pl.BlockSpec((1, H), full),
            pl.BlockSpec((H, H), full), pl.BlockSpec((1, H), full),
        ],
        out_specs=out_specs,
        out_shape=out_shape,
    )(agg, deg, x, wu, bu, wp, bp)
    return res if with_prep else (res[0], None)


def _head_body(g_ref, w2_ref, b2_ref, w3_ref, b3_ref, o_ref):
    t = _gelu(jnp.dot(g_ref[...], w2_ref[...],
                      preferred_element_type=jnp.float32) + b2_ref[...])
    o_ref[...] = jnp.dot(t, w3_ref[...],
                         preferred_element_type=jnp.float32) + b3_ref[...]


def _tc_head(g, w2, b2, w3, b3):
    rb = 512
    grid = (B // rb,)
    full = lambda i: (0, 0)
    return pl.pallas_call(
        _head_body,
        grid=grid,
        in_specs=[
            pl.BlockSpec((rb, H), lambda i: (i, 0)),
            pl.BlockSpec((H, H), full), pl.BlockSpec((1, H), full),
            pl.BlockSpec((H, C), full), pl.BlockSpec((1, C), full),
        ],
        out_specs=pl.BlockSpec((rb, C), lambda i: (i, 0)),
        out_shape=jax.ShapeDtypeStruct((B, C), jnp.float32),
    )(g, w2, b2, w3, b3)


def _zero_vmem_rows(ref, nrows, ncols):
    z = jnp.zeros((16,), jnp.float32)

    def body(i, _):
        r = i // (ncols // 16)
        c = (i % (ncols // 16)) * 16
        ref[r, pl.ds(c, 16)] = z
        return ()

    lax.fori_loop(0, nrows * (ncols // 16), body, ())


_NBUF = 2
_NGROUPS = CHUNKS_PER_TILE // _NBUF


def _edge_agg_body(h_hbm, src_hbm, dst_hbm, agg_hbm,
                   agg_sh, sidx0, sidx1, didx0, didx1, rows0, rows1,
                   sidx_t, didx_t,
                   isem0, isem1, gsem0, gsem1, ssem0, ssem1):
    sidx = (sidx0, sidx1)
    didx = (didx0, didx1)
    rows = (rows0, rows1)
    isem = (isem0, isem1)
    gsem = (gsem0, gsem1)
    ssem = (ssem0, ssem1)
    cid = lax.axis_index("c")
    sid = lax.axis_index("s")

    _zero_vmem_rows(rows0, K, H)
    for j in range(ROWS_PER_TILE // K):
        pltpu.sync_copy(rows0, agg_sh.at[pl.ds(sid * ROWS_PER_TILE + j * K, K)])
    plsc.subcore_barrier()

    base = (cid * NS + sid) * EDGES_PER_TILE

    def group(g, _):
        for b in range(_NBUF):
            off = base + (g * _NBUF + b) * K

            @pl.when(g > 0)
            def _():
                pltpu.make_async_copy(rows[b], agg_sh.at[didx[b]], ssem[b]).wait()

            pltpu.async_copy(src_hbm.at[pl.ds(off, K)], sidx[b], isem[b])
            pltpu.async_copy(dst_hbm.at[pl.ds(off, K)], didx[b], isem[b])
        for b in range(_NBUF):
            off = base + (g * _NBUF + b) * K
            pltpu.make_async_copy(src_hbm.at[pl.ds(off, K)], sidx[b], isem[b]).wait()
            pltpu.make_async_copy(dst_hbm.at[pl.ds(off, K)], didx[b], isem[b]).wait()
            pltpu.async_copy(h_hbm.at[sidx[b]], rows[b], gsem[b])
        for b in range(_NBUF):
            pltpu.make_async_copy(h_hbm.at[sidx[b]], rows[b], gsem[b]).wait()
            pltpu.async_copy(rows[b], agg_sh.at[didx[b]], ssem[b], add=True)
        return ()

    lax.fori_loop(0, _NGROUPS, group, ())
    for b in range(_NBUF):
        pltpu.make_async_copy(rows[b], agg_sh.at[didx[b]], ssem[b]).wait()
    if K_TAIL:
        off = base + CHUNKS_PER_TILE * K
        pltpu.sync_copy(src_hbm.at[pl.ds(off, K_TAIL)], sidx_t)
        pltpu.sync_copy(dst_hbm.at[pl.ds(off, K_TAIL)], didx_t)
        pltpu.sync_copy(h_hbm.at[sidx_t], rows0.at[pl.ds(0, K_TAIL)])
        pltpu.sync_copy(rows0.at[pl.ds(0, K_TAIL)], agg_sh.at[didx_t], add=True)
    plsc.subcore_barrier()

    nwb = ROWS_PER_TILE // K
    for j in range(nwb):
        r0 = sid * ROWS_PER_TILE + j * K
        rb = rows[j % _NBUF]
        sb = gsem[j % _NBUF]
        if j >= _NBUF:
            rp = sid * ROWS_PER_TILE + (j - _NBUF) * K
            pltpu.make_async_copy(rb, agg_hbm.at[cid, pl.ds(rp, K)], sb).wait()
        pltpu.sync_copy(agg_sh.at[pl.ds(r0, K)], rb)
        pltpu.async_copy(rb, agg_hbm.at[cid, pl.ds(r0, K)], sb)
    for j in range(nwb - _NBUF, nwb):
        r0 = sid * ROWS_PER_TILE + j * K
        pltpu.make_async_copy(rows[j % _NBUF], agg_hbm.at[cid, pl.ds(r0, K)],
                              gsem[j % _NBUF]).wait()


def _sc_edge_agg(h, src, dst):
    fn = pl.kernel(
        _edge_agg_body,
        out_type=jax.ShapeDtypeStruct((NC, NPAD, H), jnp.float32),
        mesh=_mesh(),
        compiler_params=pltpu.CompilerParams(needs_layout_passes=False),
        scratch_types=(
            pltpu.VMEM_SHARED((NPAD, H), jnp.float32),
            pltpu.VMEM((K,), jnp.int32),
            pltpu.VMEM((K,), jnp.int32),
            pltpu.VMEM((K,), jnp.int32),
            pltpu.VMEM((K,), jnp.int32),
            pltpu.VMEM((K, H), jnp.float32),
            pltpu.VMEM((K, H), jnp.float32),
            pltpu.VMEM((K_TAIL,), jnp.int32),
            pltpu.VMEM((K_TAIL,), jnp.int32),
            pltpu.SemaphoreType.DMA,
            pltpu.SemaphoreType.DMA,
            pltpu.SemaphoreType.DMA,
            pltpu.SemaphoreType.DMA,
            pltpu.SemaphoreType.DMA,
            pltpu.SemaphoreType.DMA,
        ),
    )
    return fn(h, src, dst)


_HR = 128
_HR_PER_TILE = _HR // NS


def _degree_body(dst_hbm, deg_hbm, stage_sh, didx_v, hist_v, acc_v, tmp_v):
    cid = lax.axis_index("c")
    sid = lax.axis_index("s")
    z = jnp.zeros((16,), jnp.float32)

    def zrow(i, _):
        hist_v[i // 8, pl.ds((i % 8) * 16, 16)] = z
        return ()

    lax.fori_loop(0, _HR * 8, zrow, ())

    base = (cid * NS + sid) * EDGES_PER_TILE
    pltpu.sync_copy(dst_hbm.at[pl.ds(base, EDGES_PER_TILE)], didx_v)
    one = jnp.ones((16,), jnp.float32)

    def step(j, _):
        idx = didx_v[pl.ds(j * 16, 16)]
        plsc.addupdate_scatter(
            hist_v, [lax.shift_right_logical(idx, 7),
                     lax.bitwise_and(idx, 127)], one)
        return ()

    lax.fori_loop(0, EDGES_PER_TILE // 16, step, ())

    pltpu.sync_copy(hist_v, stage_sh.at[sid])
    plsc.subcore_barrier()
    r0 = sid * _HR_PER_TILE

    def zacc(i, _):
        acc_v[i // 8, pl.ds((i % 8) * 16, 16)] = z
        return ()

    lax.fori_loop(0, _HR_PER_TILE * 8, zacc, ())
    for t in range(NS):
        pltpu.sync_copy(stage_sh.at[t, pl.ds(r0, _HR_PER_TILE)], tmp_v)

        def radd(i, _):
            r, c = i // 8, (i % 8) * 16
            acc_v[r, pl.ds(c, 16)] += tmp_v[r, pl.ds(c, 16)]
            return ()

        lax.fori_loop(0, _HR_PER_TILE * 8, radd, ())
    pltpu.sync_copy(acc_v, deg_hbm.at[cid, pl.ds(r0, _HR_PER_TILE)])


def _sc_degree(dst):
    fn = pl.kernel(
        _degree_body,
        out_type=jax.ShapeDtypeStruct((NC, _HR, 128), jnp.float32),
        mesh=_mesh(),
        compiler_params=pltpu.CompilerParams(needs_layout_passes=False),
        scratch_types=(
            pltpu.VMEM_SHARED((NS, _HR, 128), jnp.float32),
            pltpu.VMEM((EDGES_PER_TILE,), jnp.int32),
            pltpu.VMEM((_HR, 128), jnp.float32),
            pltpu.VMEM((_HR_PER_TILE, 128), jnp.float32),
            pltpu.VMEM((_HR_PER_TILE, 128), jnp.float32),
        ),
    )
    return fn(dst).reshape(NC, _HR * 128, 1)


def _gather_body(x_hbm, idx_hbm, out_hbm, idx_v, rows_v):
    wid = lax.axis_index("c") * NS + lax.axis_index("s")
    b0 = wid * (B // NW)
    pltpu.sync_copy(idx_hbm.at[pl.ds(b0, B // NW)], idx_v)
    pltpu.sync_copy(x_hbm.at[idx_v], rows_v)
    pltpu.sync_copy(rows_v, out_hbm.at[pl.ds(b0, B // NW)])


def _sc_gather(x, idx):
    fn = pl.kernel(
        _gather_body,
        out_type=jax.ShapeDtypeStruct((B, H), jnp.float32),
        mesh=_mesh(),
        compiler_params=pltpu.CompilerParams(needs_layout_passes=False),
        scratch_types=(
            pltpu.VMEM((B // NW,), jnp.int32),
            pltpu.VMEM((B // NW, H), jnp.float32),
        ),
    )
    return fn(x, idx)


def kernel(node_features, edges, input_node_indices,
           W1, b1, Wp0, bp0, Wu0, bu0, Wp1, bp1, Wu1, bu1,
           W2, b2, W3, b3):
    src = edges[0]
    dst = edges[1]
    r = lambda b: b.reshape(1, -1)

    x1, h0 = _tc_dense2(node_features, W1, r(b1), Wp0, r(bp0))
    deg = _sc_degree(dst)
    agg0 = _sc_edge_agg(h0, src, dst)
    x2, h1 = _tc_update(agg0, deg, x1, Wu0, r(bu0), Wp1, r(bp1), with_prep=True)
    agg1 = _sc_edge_agg(h1, src, dst)
    x3, _ = _tc_update(agg1, deg, x2, Wu1, r(bu1), Wu1, r(bu1), with_prep=False)
    g = _sc_gather(x3, input_node_indices)
    return _tc_head(g, W2, r(b2), W3, r(b3))


# speedup vs baseline: 10.1659x; 1.0906x over previous
import functools

import jax
import jax.numpy as jnp
from jax import lax
from jax.experimental import pallas as pl
from jax.experimental.pallas import tpu as pltpu
from jax.experimental.pallas import tpu_sc as plsc

N = 10000
E = 320000
D = 128
H = 128
C = 7
B = 4096

NC = 2
NS = 16
NW = NC * NS
NPAD = 10240
ROWS_PER_TILE = NPAD // NS
K = 128
EDGES_PER_TILE = E // NW
CHUNKS_PER_TILE = EDGES_PER_TILE // K
K_TAIL = EDGES_PER_TILE - CHUNKS_PER_TILE * K

_mesh = lambda: plsc.VectorSubcoreMesh(
    core_axis_name="c", subcore_axis_name="s", num_cores=NC, num_subcores=NS)


def _gelu(x):
    return jax.nn.gelu(x)


_RB = 1000


def _dense2_body(x_ref, wa_ref, ba_ref, wb_ref, bb_ref, ya_ref, yb_ref):
    ya = _gelu(jnp.dot(x_ref[...], wa_ref[...],
                       preferred_element_type=jnp.float32) + ba_ref[...])
    ya_ref[...] = ya
    yb_ref[...] = _gelu(jnp.dot(ya, wb_ref[...],
                                preferred_element_type=jnp.float32) + bb_ref[...])


def _tc_dense2(x, wa, ba, wb, bb):
    n = x.shape[0]
    grid = (n // _RB,)
    full = lambda i: (0, 0)
    return pl.pallas_call(
        _dense2_body,
        grid=grid,
        in_specs=[
            pl.BlockSpec((_RB, H), lambda i: (i, 0)),
            pl.BlockSpec((H, H), full), pl.BlockSpec((1, H), full),
            pl.BlockSpec((H, H), full), pl.BlockSpec((1, H), full),
        ],
        out_specs=[pl.BlockSpec((_RB, H), lambda i: (i, 0)),
                   pl.BlockSpec((_RB, H), lambda i: (i, 0))],
        out_shape=[jax.ShapeDtypeStruct((n, H), jnp.float32),
                   jax.ShapeDtypeStruct((n, H), jnp.float32)],
    )(x, wa, ba, wb, bb)


def _update_body(with_prep, agg_ref, deg_ref, x_ref, wu_ref, bu_ref,
                 wp_ref, bp_ref, *out_refs):
    deg = jnp.maximum(deg_ref[0, :, 0:1] + deg_ref[1, :, 0:1], 1.0)
    a = (agg_ref[0] + agg_ref[1]) * (1.0 / deg)
    x2 = _gelu(jnp.dot(a, wu_ref[...],
                       preferred_element_type=jnp.float32) + bu_ref[...]) + x_ref[...]
    out_refs[0][...] = x2
    if with_prep:
        out_refs[1][...] = _gelu(jnp.dot(x2, wp_ref[...],
                                         preferred_element_type=jnp.float32) + bp_ref[...])


def _tc_update(agg, deg, x, wu, bu, wp, bp, with_prep):
    grid = (N // _RB,)
    full = lambda i: (0, 0)
    out_specs = [pl.BlockSpec((_RB, H), lambda i: (i, 0))]
    out_shape = [jax.ShapeDtypeStruct((N, H), jnp.float32)]
    if with_prep:
        out_specs = out_specs * 2
        out_shape = out_shape * 2
    res = pl.pallas_call(
        functools.partial(_update_body, with_prep),
        grid=grid,
        in_specs=[
            pl.BlockSpec((2, _RB, H), lambda i: (0, i, 0)),
            pl.BlockSpec((2, _RB, 1), lambda i: (0, i, 0)),
            pl.BlockSpec((_RB, H), lambda i: (i, 0)),
            pl.BlockSpec((H, H), full), pl.BlockSpec((1, H), full),
            pl.BlockSpec((H, H), full), pl.BlockSpec((1, H), full),
        ],
        out_specs=out_specs,
        out_shape=out_shape,
    )(agg, deg, x, wu, bu, wp, bp)
    return res if with_prep else (res[0], None)


def _head_body(g_ref, w2_ref, b2_ref, w3_ref, b3_ref, o_ref):
    t = _gelu(jnp.dot(g_ref[...], w2_ref[...],
                      preferred_element_type=jnp.float32) + b2_ref[...])
    o_ref[...] = jnp.dot(t, w3_ref[...],
                         preferred_element_type=jnp.float32) + b3_ref[...]


def _tc_head(g, w2, b2, w3, b3):
    rb = 512
    grid = (B // rb,)
    full = lambda i: (0, 0)
    return pl.pallas_call(
        _head_body,
        grid=grid,
        in_specs=[
            pl.BlockSpec((rb, H), lambda i: (i, 0)),
            pl.BlockSpec((H, H), full), pl.BlockSpec((1, H), full),
            pl.BlockSpec((H, C), full), pl.BlockSpec((1, C), full),
        ],
        out_specs=pl.BlockSpec((rb, C), lambda i: (i, 0)),
        out_shape=jax.ShapeDtypeStruct((B, C), jnp.float32),
    )(g, w2, b2, w3, b3)


def _zero_vmem_rows(ref, nrows, ncols):
    z = jnp.zeros((16,), jnp.float32)

    def body(i, _):
        r = i // (ncols // 16)
        c = (i % (ncols // 16)) * 16
        ref[r, pl.ds(c, 16)] = z
        return ()

    lax.fori_loop(0, nrows * (ncols // 16), body, ())


_NBUF = 2
_NIDX = 4
_NGROUPS = CHUNKS_PER_TILE // _NBUF


def _edge_agg_body(h_hbm, src_hbm, dst_hbm, agg_hbm,
                   agg_sh, rows0, rows1,
                   sidx0, sidx1, sidx2, sidx3, didx0, didx1, didx2, didx3,
                   sidx_t, didx_t,
                   isem0, isem1, isem2, isem3, gsem0, gsem1, ssem0, ssem1):
    sidx = (sidx0, sidx1, sidx2, sidx3)
    didx = (didx0, didx1, didx2, didx3)
    rows = (rows0, rows1)
    isem = (isem0, isem1, isem2, isem3)
    gsem = (gsem0, gsem1)
    ssem = (ssem0, ssem1)
    cid = lax.axis_index("c")
    sid = lax.axis_index("s")

    _zero_vmem_rows(rows0, K, H)
    for j in range(ROWS_PER_TILE // K):
        pltpu.sync_copy(rows0, agg_sh.at[pl.ds(sid * ROWS_PER_TILE + j * K, K)])
    plsc.subcore_barrier()

    base = (cid * NS + sid) * EDGES_PER_TILE

    def start_idx(i, s):
        off = base + i * K
        pltpu.async_copy(src_hbm.at[pl.ds(off, K)], sidx[s], isem[s])
        pltpu.async_copy(dst_hbm.at[pl.ds(off, K)], didx[s], isem[s])

    def wait_idx(i, s):
        off = base + i * K
        pltpu.make_async_copy(src_hbm.at[pl.ds(off, K)], sidx[s], isem[s]).wait()
        pltpu.make_async_copy(dst_hbm.at[pl.ds(off, K)], didx[s], isem[s]).wait()

    def run_group(i0, s0, first, prefetch):
        for b in range(_NBUF):
            i = i0 + b
            s = s0 + b
            sp = (s + _NBUF) % _NIDX
            if not first:
                pltpu.make_async_copy(rows[b], agg_sh.at[didx[sp]], ssem[b]).wait()
            if prefetch:
                @pl.when(i + _NBUF < CHUNKS_PER_TILE)
                def _():
                    start_idx(i + _NBUF, sp)
            wait_idx(i, s)
            pltpu.async_copy(h_hbm.at[sidx[s]], rows[b], gsem[b])
        for b in range(_NBUF):
            pltpu.make_async_copy(h_hbm.at[sidx[s0 + b]], rows[b], gsem[b]).wait()
            pltpu.async_copy(rows[b], agg_sh.at[didx[s0 + b]], ssem[b], add=True)

    for i in range(_NBUF):
        start_idx(i, i)

    run_group(0, 0, True, True)
    run_group(2, 2, False, True)

    def supergroup(g, _):
        i0 = (g + 1) * (2 * _NBUF)
        run_group(i0, 0, False, True)
        run_group(i0 + _NBUF, 2, False, True)
        return ()

    lax.fori_loop(0, CHUNKS_PER_TILE // (2 * _NBUF) - 1, supergroup, ())
    run_group(CHUNKS_PER_TILE - _NBUF, 0, False, False)
    for b in range(_NBUF):
        pltpu.make_async_copy(rows[b], agg_sh.at[didx[b]], ssem[b]).wait()
    if K_TAIL:
        off = base + CHUNKS_PER_TILE * K
        pltpu.sync_copy(src_hbm.at[pl.ds(off, K_TAIL)], sidx_t)
        pltpu.sync_copy(dst_hbm.at[pl.ds(off, K_TAIL)], didx_t)
        pltpu.sync_copy(h_hbm.at[sidx_t], rows0.at[pl.ds(0, K_TAIL)])
        pltpu.sync_copy(rows0.at[pl.ds(0, K_TAIL)], agg_sh.at[didx_t], add=True)
    plsc.subcore_barrier()

    nwb = ROWS_PER_TILE // K
    for j in range(nwb):
        r0 = sid * ROWS_PER_TILE + j * K
        rb = rows[j % _NBUF]
        sb = gsem[j % _NBUF]
        if j >= _NBUF:
            rp = sid * ROWS_PER_TILE + (j - _NBUF) * K
            pltpu.make_async_copy(rb, agg_hbm.at[cid, pl.ds(rp, K)], sb).wait()
        pltpu.sync_copy(agg_sh.at[pl.ds(r0, K)], rb)
        pltpu.async_copy(rb, agg_hbm.at[cid, pl.ds(r0, K)], sb)
    for j in range(nwb - _NBUF, nwb):
        r0 = sid * ROWS_PER_TILE + j * K
        pltpu.make_async_copy(rows[j % _NBUF], agg_hbm.at[cid, pl.ds(r0, K)],
                              gsem[j % _NBUF]).wait()


def _sc_edge_agg(h, src, dst):
    fn = pl.kernel(
        _edge_agg_body,
        out_type=jax.ShapeDtypeStruct((NC, NPAD, H), jnp.float32),
        mesh=_mesh(),
        compiler_params=pltpu.CompilerParams(needs_layout_passes=False),
        scratch_types=(
            pltpu.VMEM_SHARED((NPAD, H), jnp.float32),
            pltpu.VMEM((K, H), jnp.float32),
            pltpu.VMEM((K, H), jnp.float32),
            pltpu.VMEM((K,), jnp.int32),
            pltpu.VMEM((K,), jnp.int32),
            pltpu.VMEM((K,), jnp.int32),
            pltpu.VMEM((K,), jnp.int32),
            pltpu.VMEM((K,), jnp.int32),
            pltpu.VMEM((K,), jnp.int32),
            pltpu.VMEM((K,), jnp.int32),
            pltpu.VMEM((K,), jnp.int32),
            pltpu.VMEM((K_TAIL,), jnp.int32),
            pltpu.VMEM((K_TAIL,), jnp.int32),
            pltpu.SemaphoreType.DMA,
            pltpu.SemaphoreType.DMA,
            pltpu.SemaphoreType.DMA,
            pltpu.SemaphoreType.DMA,
            pltpu.SemaphoreType.DMA,
            pltpu.SemaphoreType.DMA,
            pltpu.SemaphoreType.DMA,
            pltpu.SemaphoreType.DMA,
        ),
    )
    return fn(h, src, dst)


_HR = 128
_HR_PER_TILE = _HR // NS


def _degree_body(dst_hbm, deg_hbm, stage_sh, didx_v, hist_v, acc_v, tmp_v):
    cid = lax.axis_index("c")
    sid = lax.axis_index("s")
    z = jnp.zeros((16,), jnp.float32)

    def zrow(i, _):
        hist_v[i // 8, pl.ds((i % 8) * 16, 16)] = z
        return ()

    lax.fori_loop(0, _HR * 8, zrow, ())

    base = (cid * NS + sid) * EDGES_PER_TILE
    pltpu.sync_copy(dst_hbm.at[pl.ds(base, EDGES_PER_TILE)], didx_v)
    one = jnp.ones((16,), jnp.float32)

    def step(j, _):
        idx = didx_v[pl.ds(j * 16, 16)]
        plsc.addupdate_scatter(
            hist_v, [lax.shift_right_logical(idx, 7),
                     lax.bitwise_and(idx, 127)], one)
        return ()

    lax.fori_loop(0, EDGES_PER_TILE // 16, step, ())

    pltpu.sync_copy(hist_v, stage_sh.at[sid])
    plsc.subcore_barrier()
    r0 = sid * _HR_PER_TILE

    def zacc(i, _):
        acc_v[i // 8, pl.ds((i % 8) * 16, 16)] = z
        return ()

    lax.fori_loop(0, _HR_PER_TILE * 8, zacc, ())
    for t in range(NS):
        pltpu.sync_copy(stage_sh.at[t, pl.ds(r0, _HR_PER_TILE)], tmp_v)

        def radd(i, _):
            r, c = i // 8, (i % 8) * 16
            acc_v[r, pl.ds(c, 16)] += tmp_v[r, pl.ds(c, 16)]
            return ()

        lax.fori_loop(0, _HR_PER_TILE * 8, radd, ())
    pltpu.sync_copy(acc_v, deg_hbm.at[cid, pl.ds(r0, _HR_PER_TILE)])


def _sc_degree(dst):
    fn = pl.kernel(
        _degree_body,
        out_type=jax.ShapeDtypeStruct((NC, _HR, 128), jnp.float32),
        mesh=_mesh(),
        compiler_params=pltpu.CompilerParams(needs_layout_passes=False),
        scratch_types=(
            pltpu.VMEM_SHARED((NS, _HR, 128), jnp.float32),
            pltpu.VMEM((EDGES_PER_TILE,), jnp.int32),
            pltpu.VMEM((_HR, 128), jnp.float32),
            pltpu.VMEM((_HR_PER_TILE, 128), jnp.float32),
            pltpu.VMEM((_HR_PER_TILE, 128), jnp.float32),
        ),
    )
    return fn(dst).reshape(NC, _HR * 128, 1)


def _gather_body(x_hbm, idx_hbm, out_hbm, idx_v, rows_v):
    wid = lax.axis_index("c") * NS + lax.axis_index("s")
    b0 = wid * (B // NW)
    pltpu.sync_copy(idx_hbm.at[pl.ds(b0, B // NW)], idx_v)
    pltpu.sync_copy(x_hbm.at[idx_v], rows_v)
    pltpu.sync_copy(rows_v, out_hbm.at[pl.ds(b0, B // NW)])


def _sc_gather(x, idx):
    fn = pl.kernel(
        _gather_body,
        out_type=jax.ShapeDtypeStruct((B, H), jnp.float32),
        mesh=_mesh(),
        compiler_params=pltpu.CompilerParams(needs_layout_passes=False),
        scratch_types=(
            pltpu.VMEM((B // NW,), jnp.int32),
            pltpu.VMEM((B // NW, H), jnp.float32),
        ),
    )
    return fn(x, idx)


def kernel(node_features, edges, input_node_indices,
           W1, b1, Wp0, bp0, Wu0, bu0, Wp1, bp1, Wu1, bu1,
           W2, b2, W3, b3):
    src = edges[0]
    dst = edges[1]
    r = lambda b: b.reshape(1, -1)

    x1, h0 = _tc_dense2(node_features, W1, r(b1), Wp0, r(bp0))
    deg = _sc_degree(dst)
    agg0 = _sc_edge_agg(h0, src, dst)
    x2, h1 = _tc_update(agg0, deg, x1, Wu0, r(bu0), Wp1, r(bp1), with_prep=True)
    agg1 = _sc_edge_agg(h1, src, dst)
    x3, _ = _tc_update(agg1, deg, x2, Wu1, r(bu1), Wu1, r(bu1), with_prep=False)
    g = _sc_gather(x3, input_node_indices)
    return _tc_head(g, W2, r(b2), W3, r(b3))


# speedup vs baseline: 10.1757x; 1.0010x over previous
import functools

import jax
import jax.numpy as jnp
from jax import lax
from jax.experimental import pallas as pl
from jax.experimental.pallas import tpu as pltpu
from jax.experimental.pallas import tpu_sc as plsc

N = 10000
E = 320000
D = 128
H = 128
C = 7
B = 4096

NC = 2
NS = 16
NW = NC * NS
NPAD = 10240
ROWS_PER_TILE = NPAD // NS
K = 128
EDGES_PER_TILE = E // NW
CHUNKS_PER_TILE = EDGES_PER_TILE // K
K_TAIL = EDGES_PER_TILE - CHUNKS_PER_TILE * K

_mesh = lambda: plsc.VectorSubcoreMesh(
    core_axis_name="c", subcore_axis_name="s", num_cores=NC, num_subcores=NS)


def _gelu(x):
    return jax.nn.gelu(x)


_RB = 1000


def _dense2_body(x_ref, wa_ref, ba_ref, wb_ref, bb_ref, ya_ref, yb_ref):
    ya = _gelu(jnp.dot(x_ref[...], wa_ref[...],
                       preferred_element_type=jnp.float32) + ba_ref[...])
    ya_ref[...] = ya
    yb_ref[...] = _gelu(jnp.dot(ya, wb_ref[...],
                                preferred_element_type=jnp.float32) + bb_ref[...])


def _tc_dense2(x, wa, ba, wb, bb):
    n = x.shape[0]
    grid = (n // _RB,)
    full = lambda i: (0, 0)
    return pl.pallas_call(
        _dense2_body,
        grid=grid,
        in_specs=[
            pl.BlockSpec((_RB, H), lambda i: (i, 0)),
            pl.BlockSpec((H, H), full), pl.BlockSpec((1, H), full),
            pl.BlockSpec((H, H), full), pl.BlockSpec((1, H), full),
        ],
        out_specs=[pl.BlockSpec((_RB, H), lambda i: (i, 0)),
                   pl.BlockSpec((_RB, H), lambda i: (i, 0))],
        out_shape=[jax.ShapeDtypeStruct((n, H), jnp.float32),
                   jax.ShapeDtypeStruct((n, H), jnp.float32)],
    )(x, wa, ba, wb, bb)


def _update_body(with_prep, agg_ref, deg_ref, x_ref, wu_ref, bu_ref,
                 wp_ref, bp_ref, *out_refs):
    deg = jnp.maximum(deg_ref[0, :, 0:1] + deg_ref[1, :, 0:1], 1.0)
    a = (agg_ref[0] + agg_ref[1]) * (1.0 / deg)
    x2 = _gelu(jnp.dot(a, wu_ref[...],
                       preferred_element_type=jnp.float32) + bu_ref[...]) + x_ref[...]
    out_refs[0][...] = x2
    if with_prep:
        out_refs[1][...] = _gelu(jnp.dot(x2, wp_ref[...],
                                         preferred_element_type=jnp.float32) + bp_ref[...])


def _tc_update(agg, deg, x, wu, bu, wp, bp, with_prep):
    grid = (N // _RB,)
    full = lambda i: (0, 0)
    out_specs = [pl.BlockSpec((_RB, H), lambda i: (i, 0))]
    out_shape = [jax.ShapeDtypeStruct((N, H), jnp.float32)]
    if with_prep:
        out_specs = out_specs * 2
        out_shape = out_shape * 2
    res = pl.pallas_call(
        functools.partial(_update_body, with_prep),
        grid=grid,
        in_specs=[
            pl.BlockSpec((2, _RB, H), lambda i: (0, i, 0)),
            pl.BlockSpec((2, _RB, 1), lambda i: (0, i, 0)),
            pl.BlockSpec((_RB, H), lambda i: (i, 0)),
            pl.BlockSpec((H, H), full), pl.BlockSpec((1, H), full),
            pl.BlockSpec((H, H), full), pl.BlockSpec((1, H), full),
        ],
        out_specs=out_specs,
        out_shape=out_shape,
    )(agg, deg, x, wu, bu, wp, bp)
    return res if with_prep else (res[0], None)


def _head_body(g_ref, w2_ref, b2_ref, w3_ref, b3_ref, o_ref):
    t = _gelu(jnp.dot(g_ref[...], w2_ref[...],
                      preferred_element_type=jnp.float32) + b2_ref[...])
    o_ref[...] = jnp.dot(t, w3_ref[...],
                         preferred_element_type=jnp.float32) + b3_ref[...]


def _tc_head(g, w2, b2, w3, b3):
    rb = 512
    grid = (B // rb,)
    full = lambda i: (0, 0)
    return pl.pallas_call(
        _head_body,
        grid=grid,
        in_specs=[
            pl.BlockSpec((rb, H), lambda i: (i, 0)),
            pl.BlockSpec((H, H), full), pl.BlockSpec((1, H), full),
            pl.BlockSpec((H, C), full), pl.BlockSpec((1, C), full),
        ],
        out_specs=pl.BlockSpec((rb, C), lambda i: (i, 0)),
        out_shape=jax.ShapeDtypeStruct((B, C), jnp.float32),
    )(g, w2, b2, w3, b3)


def _zero_vmem_rows(ref, nrows, ncols):
    z = jnp.zeros((16,), jnp.float32)

    def body(i, _):
        r = i // (ncols // 16)
        c = (i % (ncols // 16)) * 16
        ref[r, pl.ds(c, 16)] = z
        return ()

    lax.fori_loop(0, nrows * (ncols // 16), body, ())


_NBUF = 2
_NIDX = 4
_NGROUPS = CHUNKS_PER_TILE // _NBUF


def _edge_agg_body(h_hbm, edges_hbm, agg_hbm,
                   agg_sh, rows0, rows1,
                   eidx0, eidx1, eidx2, eidx3,
                   isem0, isem1, isem2, isem3, gsem0, gsem1, ssem0, ssem1):
    eidx = (eidx0, eidx1, eidx2, eidx3)
    sidx = tuple(e.at[0] for e in eidx)
    didx = tuple(e.at[1] for e in eidx)
    rows = (rows0, rows1)
    isem = (isem0, isem1, isem2, isem3)
    gsem = (gsem0, gsem1)
    ssem = (ssem0, ssem1)
    cid = lax.axis_index("c")
    sid = lax.axis_index("s")

    _zero_vmem_rows(rows0, K, H)
    for j in range(ROWS_PER_TILE // K):
        pltpu.sync_copy(rows0, agg_sh.at[pl.ds(sid * ROWS_PER_TILE + j * K, K)])
    plsc.subcore_barrier()

    wid = cid * NS + sid
    base = (wid * CHUNKS_PER_TILE + jnp.minimum(wid, E // K - NW * CHUNKS_PER_TILE)) * K

    def start_idx(i, s):
        off = base + i * K
        pltpu.async_copy(edges_hbm.at[:, pl.ds(off, K)], eidx[s], isem[s])

    def wait_idx(i, s):
        off = base + i * K
        pltpu.make_async_copy(edges_hbm.at[:, pl.ds(off, K)], eidx[s], isem[s]).wait()

    def run_group(i0, s0, first, prefetch):
        for b in range(_NBUF):
            i = i0 + b
            s = s0 + b
            sp = (s + _NBUF) % _NIDX
            if not first:
                pltpu.make_async_copy(rows[b], agg_sh.at[didx[sp]], ssem[b]).wait()
            if prefetch:
                @pl.when(i + _NBUF < CHUNKS_PER_TILE)
                def _():
                    start_idx(i + _NBUF, sp)
            wait_idx(i, s)
            pltpu.async_copy(h_hbm.at[sidx[s]], rows[b], gsem[b])
        for b in range(_NBUF):
            pltpu.make_async_copy(h_hbm.at[sidx[s0 + b]], rows[b], gsem[b]).wait()
            pltpu.async_copy(rows[b], agg_sh.at[didx[s0 + b]], ssem[b], add=True)

    for i in range(_NBUF):
        start_idx(i, i)

    run_group(0, 0, True, True)
    run_group(2, 2, False, True)

    def supergroup(g, _):
        i0 = (g + 1) * (2 * _NBUF)
        run_group(i0, 0, False, True)
        run_group(i0 + _NBUF, 2, False, True)
        return ()

    lax.fori_loop(0, CHUNKS_PER_TILE // (2 * _NBUF) - 1, supergroup, ())
    run_group(CHUNKS_PER_TILE - _NBUF, 0, False, False)
    for b in range(_NBUF):
        pltpu.make_async_copy(rows[b], agg_sh.at[didx[b]], ssem[b]).wait()
    @pl.when(wid < E // K - NW * CHUNKS_PER_TILE)
    def _():
        off = (wid * (CHUNKS_PER_TILE + 1) + CHUNKS_PER_TILE) * K
        pltpu.sync_copy(edges_hbm.at[:, pl.ds(off, K)], eidx[0])
        pltpu.sync_copy(h_hbm.at[sidx[0]], rows0)
        pltpu.sync_copy(rows0, agg_sh.at[didx[0]], add=True)
    plsc.subcore_barrier()

    nwb = ROWS_PER_TILE // K
    for j in range(nwb):
        r0 = sid * ROWS_PER_TILE + j * K
        rb = rows[j % _NBUF]
        sb = gsem[j % _NBUF]
        if j >= _NBUF:
            rp = sid * ROWS_PER_TILE + (j - _NBUF) * K
            pltpu.make_async_copy(rb, agg_hbm.at[cid, pl.ds(rp, K)], sb).wait()
        pltpu.sync_copy(agg_sh.at[pl.ds(r0, K)], rb)
        pltpu.async_copy(rb, agg_hbm.at[cid, pl.ds(r0, K)], sb)
    for j in range(nwb - _NBUF, nwb):
        r0 = sid * ROWS_PER_TILE + j * K
        pltpu.make_async_copy(rows[j % _NBUF], agg_hbm.at[cid, pl.ds(r0, K)],
                              gsem[j % _NBUF]).wait()


def _sc_edge_agg(h, edges):
    fn = pl.kernel(
        _edge_agg_body,
        out_type=jax.ShapeDtypeStruct((NC, NPAD, H), jnp.float32),
        mesh=_mesh(),
        compiler_params=pltpu.CompilerParams(needs_layout_passes=False),
        scratch_types=(
            pltpu.VMEM_SHARED((NPAD, H), jnp.float32),
            pltpu.VMEM((K, H), jnp.float32),
            pltpu.VMEM((K, H), jnp.float32),
            pltpu.VMEM((2, K), jnp.int32),
            pltpu.VMEM((2, K), jnp.int32),
            pltpu.VMEM((2, K), jnp.int32),
            pltpu.VMEM((2, K), jnp.int32),
            pltpu.SemaphoreType.DMA,
            pltpu.SemaphoreType.DMA,
            pltpu.SemaphoreType.DMA,
            pltpu.SemaphoreType.DMA,
            pltpu.SemaphoreType.DMA,
            pltpu.SemaphoreType.DMA,
            pltpu.SemaphoreType.DMA,
            pltpu.SemaphoreType.DMA,
        ),
    )
    return fn(h, edges)


_HR = 128
_HR_PER_TILE = _HR // NS


def _degree_body(dst_hbm, deg_hbm, stage_sh, didx_v, hist_v, acc_v, tmp_v):
    cid = lax.axis_index("c")
    sid = lax.axis_index("s")
    z = jnp.zeros((16,), jnp.float32)

    def zrow(i, _):
        hist_v[i // 8, pl.ds((i % 8) * 16, 16)] = z
        return ()

    lax.fori_loop(0, _HR * 8, zrow, ())

    base = (cid * NS + sid) * EDGES_PER_TILE
    pltpu.sync_copy(dst_hbm.at[pl.ds(base, EDGES_PER_TILE)], didx_v)
    one = jnp.ones((16,), jnp.float32)

    def step(j, _):
        idx = didx_v[pl.ds(j * 16, 16)]
        plsc.addupdate_scatter(
            hist_v, [lax.shift_right_logical(idx, 7),
                     lax.bitwise_and(idx, 127)], one)
        return ()

    lax.fori_loop(0, EDGES_PER_TILE // 16, step, ())

    pltpu.sync_copy(hist_v, stage_sh.at[sid])
    plsc.subcore_barrier()
    r0 = sid * _HR_PER_TILE

    def zacc(i, _):
        acc_v[i // 8, pl.ds((i % 8) * 16, 16)] = z
        return ()

    lax.fori_loop(0, _HR_PER_TILE * 8, zacc, ())
    for t in range(NS):
        pltpu.sync_copy(stage_sh.at[t, pl.ds(r0, _HR_PER_TILE)], tmp_v)

        def radd(i, _):
            r, c = i // 8, (i % 8) * 16
            acc_v[r, pl.ds(c, 16)] += tmp_v[r, pl.ds(c, 16)]
            return ()

        lax.fori_loop(0, _HR_PER_TILE * 8, radd, ())
    pltpu.sync_copy(acc_v, deg_hbm.at[cid, pl.ds(r0, _HR_PER_TILE)])


def _sc_degree(dst):
    fn = pl.kernel(
        _degree_body,
        out_type=jax.ShapeDtypeStruct((NC, _HR, 128), jnp.float32),
        mesh=_mesh(),
        compiler_params=pltpu.CompilerParams(needs_layout_passes=False),
        scratch_types=(
            pltpu.VMEM_SHARED((NS, _HR, 128), jnp.float32),
            pltpu.VMEM((EDGES_PER_TILE,), jnp.int32),
            pltpu.VMEM((_HR, 128), jnp.float32),
            pltpu.VMEM((_HR_PER_TILE, 128), jnp.float32),
            pltpu.VMEM((_HR_PER_TILE, 128), jnp.float32),
        ),
    )
    return fn(dst).reshape(NC, _HR * 128, 1)


def _gather_body(x_hbm, idx_hbm, out_hbm, idx_v, rows_v):
    wid = lax.axis_index("c") * NS + lax.axis_index("s")
    b0 = wid * (B // NW)
    pltpu.sync_copy(idx_hbm.at[pl.ds(b0, B // NW)], idx_v)
    pltpu.sync_copy(x_hbm.at[idx_v], rows_v)
    pltpu.sync_copy(rows_v, out_hbm.at[pl.ds(b0, B // NW)])


def _sc_gather(x, idx):
    fn = pl.kernel(
        _gather_body,
        out_type=jax.ShapeDtypeStruct((B, H), jnp.float32),
        mesh=_mesh(),
        compiler_params=pltpu.CompilerParams(needs_layout_passes=False),
        scratch_types=(
            pltpu.VMEM((B // NW,), jnp.int32),
            pltpu.VMEM((B // NW, H), jnp.float32),
        ),
    )
    return fn(x, idx)


def kernel(node_features, edges, input_node_indices,
           W1, b1, Wp0, bp0, Wu0, bu0, Wp1, bp1, Wu1, bu1,
           W2, b2, W3, b3):
    dst = edges[1]
    r = lambda b: b.reshape(1, -1)

    x1, h0 = _tc_dense2(node_features, W1, r(b1), Wp0, r(bp0))
    deg = _sc_degree(dst)
    agg0 = _sc_edge_agg(h0, edges)
    x2, h1 = _tc_update(agg0, deg, x1, Wu0, r(bu0), Wp1, r(bp1), with_prep=True)
    agg1 = _sc_edge_agg(h1, edges)
    x3, _ = _tc_update(agg1, deg, x2, Wu1, r(bu1), Wu1, r(bu1), with_prep=False)
    g = _sc_gather(x3, input_node_indices)
    return _tc_head(g, W2, r(b2), W3, r(b3))
